# baseline (speedup 1.0000x reference)
import functools

import jax
import jax.numpy as jnp
from jax import lax
from jax.experimental import pallas as pl
from jax.experimental.pallas import tpu as pltpu
from jax.experimental.pallas import tpu_sc as plsc

_N = 10000
_D_IN = 128
_H1 = 64
_ENC = 32

_NC = 2
_NS = 16
_NW = _NC * _NS
_K = 128
_N_PAD = 10112
_RPT = _N_PAD // _NS

def _fill(ref, n_rows, n_vec, value):
  def body(i, _):
    for k in range(n_vec):
      ref[i, pl.ds(k * 16, 16)] = jnp.full((16,), value, jnp.float32)
    return 0
  lax.fori_loop(0, n_rows, body, 0)


def _make_seg_sum(depth, n0, n1):
  gather = depth > 16

  def body(*refs):
    if gather:
      (src3, dst3, table, out, idx_s, idx_d, rows_a, rows_b, zrow, acc,
       sem_a, sem_b) = refs
    else:
      dst3, out, idx_d, rows_a, zrow, acc, sem_a = refs
    c = lax.axis_index("c")
    s = lax.axis_index("s")
    wid = c * _NS + s
    n_vec = depth // 16

    _fill(zrow, _RPT, n_vec, 0.0)
    pltpu.sync_copy(zrow, acc.at[pl.ds(s * _RPT, _RPT)])
    if gather:
      pltpu.sync_copy(src3.at[wid], idx_s)
    else:
      _fill(rows_a, _K, n_vec, 1.0)
    pltpu.sync_copy(dst3.at[wid], idx_d)
    plsc.subcore_barrier()

    if gather:
      def run_pipeline(n_chunks):
        n_half = n_chunks // 2
        pltpu.async_copy(table.at[idx_s.at[0]], rows_a, sem_a)

        def edge_body(jj, _):
          j0 = 2 * jj
          pltpu.make_async_copy(table.at[idx_s.at[0]], rows_a, sem_a).wait()
          pltpu.async_copy(table.at[idx_s.at[j0 + 1]], rows_b, sem_b)
          pltpu.sync_copy(rows_a, acc.at[idx_d.at[j0]], add=True)
          pltpu.make_async_copy(table.at[idx_s.at[0]], rows_b, sem_b).wait()
          @pl.when(jj < n_half - 1)
          def _():
            pltpu.async_copy(table.at[idx_s.at[j0 + 2]], rows_a, sem_a)
          pltpu.sync_copy(rows_b, acc.at[idx_d.at[j0 + 1]], add=True)
          return 0
        lax.fori_loop(0, n_half, edge_body, 0)

      if n0 == n1:
        run_pipeline(n0)
      else:
        pl.when(c == 0)(lambda: run_pipeline(n0))
        pl.when(c != 0)(lambda: run_pipeline(n1))
    else:
      def run_scatter(n_chunks):
        def edge_body(j, _):
          pltpu.sync_copy(rows_a, acc.at[idx_d.at[j]], add=True)
          return 0
        lax.fori_loop(0, n_chunks, edge_body, 0)

      if n0 == n1:
        run_scatter(n0)
      else:
        pl.when(c == 0)(lambda: run_scatter(n0))
        pl.when(c != 0)(lambda: run_scatter(n1))

    plsc.subcore_barrier()
    pltpu.sync_copy(acc.at[pl.ds(s * _RPT, _RPT)],
                    out.at[c, pl.ds(s * _RPT, _RPT)])

  n_max = max(n0, n1)
  scratch = []
  if gather:
    scratch.append(pltpu.VMEM((n_max, _K), jnp.int32))
  scratch += [
      pltpu.VMEM((n_max, _K), jnp.int32),
      pltpu.VMEM((_K, depth), jnp.float32),
  ]
  if gather:
    scratch.append(pltpu.VMEM((_K, depth), jnp.float32))
  scratch += [
      pltpu.VMEM((_RPT, depth), jnp.float32),
      pltpu.VMEM_SHARED((_N_PAD, depth), jnp.float32),
      pltpu.SemaphoreType.DMA,
  ]
  if gather:
    scratch.append(pltpu.SemaphoreType.DMA)
  mesh = plsc.VectorSubcoreMesh(
      core_axis_name="c", subcore_axis_name="s", num_cores=_NC,
      num_subcores=_NS)
  return pl.kernel(
      body,
      out_type=jax.ShapeDtypeStruct((_NC, _N_PAD, depth), jnp.float32),
      mesh=mesh,
      scratch_types=scratch,
      compiler_params=pltpu.CompilerParams(use_tc_tiling_on_sc=False),
  )


def _enc1_body(x_ref, w_ref, d_ref, xw_ref, dinv_ref):
  d = d_ref[...]
  deg = d[0, :, 0:1] + d[1, :, 0:1] + 1.0
  dinv = 1.0 / jnp.sqrt(deg)
  xw = jnp.dot(x_ref[...], w_ref[...], preferred_element_type=jnp.float32)
  xw_ref[...] = xw * dinv
  dinv_ref[...] = jnp.broadcast_to(dinv, (_N_PAD, 16))


def _enc2_body(s1_ref, xwp_ref, dinv_ref, b1_ref, w2_ref, t2_ref):
  dv = dinv_ref[:, 0:1]
  s1 = s1_ref[...]
  h = dv * (s1[0] + s1[1] + xwp_ref[...]) + b1_ref[...]
  t2_ref[...] = dv * jnp.dot(h, w2_ref[...], preferred_element_type=jnp.float32)


_BM = 200


def _dec_body(s2_ref, t2_ref, dinv_ref, b2_ref, y_ref, z_ref, zt_ref):
  i = pl.program_id(0)

  @pl.when(i == 0)
  def _():
    dv = dinv_ref[:, 0:1]
    s2 = s2_ref[...]
    z = dv * (s2[0] + s2[1] + t2_ref[...]) + b2_ref[...]
    z_ref[...] = z
    zt_ref[...] = z[:_N].T

  zb = z_ref[pl.ds(i * _BM, _BM), :]
  acc = jnp.dot(zb, zt_ref[...], preferred_element_type=jnp.float32)
  y_ref[...] = 1.0 / (1.0 + jnp.exp(-acc))


_C0_SHARE = 0.675


def kernel(X, edge_index, W1, b1, W2, b2):
  E = edge_index.shape[1]
  unit = _NW * _K * 2
  e_pad = unit * ((E + unit - 1) // unit)
  n_tot = e_pad // (_NS * _K)
  n0 = 2 * int(round(_C0_SHARE * n_tot / 2))
  n0 = min(max(n0, 2), n_tot - 2)
  n1 = n_tot - n0

  ei_pad = jnp.pad(edge_index, ((0, 0), (0, e_pad - E)), constant_values=_N)
  chunks = ei_pad.reshape(2, _NS * n_tot, _K)
  c0 = chunks[:, : _NS * n0].reshape(2, _NS, n0, _K)
  c1 = chunks[:, _NS * n0 :].reshape(2, _NS, n1, _K)
  c1 = jnp.pad(c1, ((0, 0), (0, 0), (0, n0 - n1), (0, 0)),
               constant_values=_N)
  ei3 = jnp.concatenate([c0, c1], axis=1)
  src3, dst3 = ei3[0], ei3[1]
  x_pad = jnp.pad(X, ((0, _N_PAD - _N), (0, 0)))

  deg = _make_seg_sum(16, n0, n1)(dst3)

  xwp, dinv = pl.pallas_call(
      _enc1_body,
      out_shape=(
          jax.ShapeDtypeStruct((_N_PAD, _H1), jnp.float32),
          jax.ShapeDtypeStruct((_N_PAD, 16), jnp.float32),
      ),
  )(x_pad, W1, deg)

  s1 = _make_seg_sum(_H1, n0, n1)(src3, dst3, xwp)

  t2 = pl.pallas_call(
      _enc2_body,
      out_shape=jax.ShapeDtypeStruct((_N_PAD, _ENC), jnp.float32),
  )(s1, xwp, dinv, b1.reshape(1, _H1), W2)

  s2 = _make_seg_sum(_ENC, n0, n1)(src3, dst3, t2)

  y = pl.pallas_call(
      _dec_body,
      grid=(_N // _BM,),
      in_specs=[
          pl.BlockSpec((_NC, _N_PAD, _ENC), lambda i: (0, 0, 0)),
          pl.BlockSpec((_N_PAD, _ENC), lambda i: (0, 0)),
          pl.BlockSpec((_N_PAD, 16), lambda i: (0, 0)),
          pl.BlockSpec((1, _ENC), lambda i: (0, 0)),
      ],
      out_specs=pl.BlockSpec((_BM, _N), lambda i: (i, 0)),
      out_shape=jax.ShapeDtypeStruct((_N, _N), jnp.float32),
      scratch_shapes=[
          pltpu.VMEM((_N_PAD, _ENC), jnp.float32),
          pltpu.VMEM((_ENC, _N), jnp.float32),
      ],
  )(s2, t2, dinv, b2.reshape(1, _ENC))
  return y


# speedup vs baseline: 1.0514x; 1.0514x over previous
import functools

import jax
import jax.numpy as jnp
from jax import lax
from jax.experimental import pallas as pl
from jax.experimental.pallas import tpu as pltpu
from jax.experimental.pallas import tpu_sc as plsc

_N = 10000
_D_IN = 128
_H1 = 64
_ENC = 32

_NC = 2
_NS = 16
_NW = _NC * _NS
_K = 128
_N_PAD = 10112
_RPT = _N_PAD // _NS

def _fill(ref, n_rows, n_vec, value):
  def body(i, _):
    for k in range(n_vec):
      ref[i, pl.ds(k * 16, 16)] = jnp.full((16,), value, jnp.float32)
    return 0
  lax.fori_loop(0, n_rows, body, 0)


def _make_seg_sum(depth, n0, n1):
  gather = depth > 16

  def body(*refs):
    if gather:
      (src3, dst3, table, out, idx_s, idx_d, rows_a, rows_b, zrow, acc,
       sem_a, sem_b) = refs
    else:
      dst3, out, idx_d, rows_a, zrow, acc, sem_a = refs
    c = lax.axis_index("c")
    s = lax.axis_index("s")
    wid = c * _NS + s
    n_vec = depth // 16

    _fill(zrow, _RPT, n_vec, 0.0)
    pltpu.sync_copy(zrow, acc.at[pl.ds(s * _RPT, _RPT)])
    if gather:
      pltpu.sync_copy(src3.at[wid], idx_s)
    else:
      _fill(rows_a, _K, n_vec, 1.0)
    pltpu.sync_copy(dst3.at[wid], idx_d)
    plsc.subcore_barrier()

    if gather:
      def run_pipeline(n_chunks):
        n_half = n_chunks // 2
        pltpu.async_copy(table.at[idx_s.at[0]], rows_a, sem_a)

        def edge_body(jj, _):
          j0 = 2 * jj
          pltpu.make_async_copy(table.at[idx_s.at[0]], rows_a, sem_a).wait()
          pltpu.async_copy(table.at[idx_s.at[j0 + 1]], rows_b, sem_b)
          pltpu.sync_copy(rows_a, acc.at[idx_d.at[j0]], add=True)
          pltpu.make_async_copy(table.at[idx_s.at[0]], rows_b, sem_b).wait()
          @pl.when(jj < n_half - 1)
          def _():
            pltpu.async_copy(table.at[idx_s.at[j0 + 2]], rows_a, sem_a)
          pltpu.sync_copy(rows_b, acc.at[idx_d.at[j0 + 1]], add=True)
          return 0
        lax.fori_loop(0, n_half, edge_body, 0)

      if n0 == n1:
        run_pipeline(n0)
      else:
        pl.when(c == 0)(lambda: run_pipeline(n0))
        pl.when(c != 0)(lambda: run_pipeline(n1))
    else:
      def run_scatter(n_chunks):
        def edge_body(j, _):
          pltpu.sync_copy(rows_a, acc.at[idx_d.at[j]], add=True)
          return 0
        lax.fori_loop(0, n_chunks, edge_body, 0)

      if n0 == n1:
        run_scatter(n0)
      else:
        pl.when(c == 0)(lambda: run_scatter(n0))
        pl.when(c != 0)(lambda: run_scatter(n1))

    plsc.subcore_barrier()
    pltpu.sync_copy(acc.at[pl.ds(s * _RPT, _RPT)],
                    out.at[c, pl.ds(s * _RPT, _RPT)])

  n_max = max(n0, n1)
  scratch = []
  if gather:
    scratch.append(pltpu.VMEM((n_max, _K), jnp.int32))
  scratch += [
      pltpu.VMEM((n_max, _K), jnp.int32),
      pltpu.VMEM((_K, depth), jnp.float32),
  ]
  if gather:
    scratch.append(pltpu.VMEM((_K, depth), jnp.float32))
  scratch += [
      pltpu.VMEM((_RPT, depth), jnp.float32),
      pltpu.VMEM_SHARED((_N_PAD, depth), jnp.float32),
      pltpu.SemaphoreType.DMA,
  ]
  if gather:
    scratch.append(pltpu.SemaphoreType.DMA)
  mesh = plsc.VectorSubcoreMesh(
      core_axis_name="c", subcore_axis_name="s", num_cores=_NC,
      num_subcores=_NS)
  return pl.kernel(
      body,
      out_type=jax.ShapeDtypeStruct((_NC, _N_PAD, depth), jnp.float32),
      mesh=mesh,
      scratch_types=scratch,
      compiler_params=pltpu.CompilerParams(use_tc_tiling_on_sc=False),
  )


def _enc1_body(x_ref, w_ref, d_ref, xw_ref, dinv_ref):
  d = d_ref[...]
  deg = d[0, :, 0:1] + d[1, :, 0:1] + 1.0
  dinv = 1.0 / jnp.sqrt(deg)
  xw = jnp.dot(x_ref[...], w_ref[...], preferred_element_type=jnp.float32)
  xw_ref[...] = xw * dinv
  dinv_ref[...] = jnp.broadcast_to(dinv, (_N_PAD, 16))


def _enc2_body(s1_ref, xwp_ref, dinv_ref, b1_ref, w2_ref, t2_ref):
  dv = dinv_ref[:, 0:1]
  s1 = s1_ref[...]
  h = dv * (s1[0] + s1[1] + xwp_ref[...]) + b1_ref[...]
  t2_ref[...] = dv * jnp.dot(h, w2_ref[...], preferred_element_type=jnp.float32)


_BM = 400


def _zfin_body(s2_ref, t2_ref, dinv_ref, b2_ref, z_ref, zt_ref):
  dv = dinv_ref[:, 0:1]
  s2 = s2_ref[...]
  z = dv * (s2[0] + s2[1] + t2_ref[...]) + b2_ref[...]
  z = z[:_N]
  z_ref[...] = z
  zt_ref[...] = z.T


def _dec_body(zb_ref, zt_ref, y_ref):
  acc = jnp.dot(zb_ref[...], zt_ref[...], preferred_element_type=jnp.float32)
  y_ref[...] = 1.0 / (1.0 + jnp.exp(-acc))


_C0_SHARE = 0.675


def kernel(X, edge_index, W1, b1, W2, b2):
  E = edge_index.shape[1]
  unit = _NW * _K * 2
  e_pad = unit * ((E + unit - 1) // unit)
  n_tot = e_pad // (_NS * _K)
  n0 = 2 * int(round(_C0_SHARE * n_tot / 2))
  n0 = min(max(n0, 2), n_tot - 2)
  n1 = n_tot - n0

  ei_pad = jnp.pad(edge_index, ((0, 0), (0, e_pad - E)), constant_values=_N)
  chunks = ei_pad.reshape(2, _NS * n_tot, _K)
  c0 = chunks[:, : _NS * n0].reshape(2, _NS, n0, _K)
  c1 = chunks[:, _NS * n0 :].reshape(2, _NS, n1, _K)
  c1 = jnp.pad(c1, ((0, 0), (0, 0), (0, n0 - n1), (0, 0)),
               constant_values=_N)
  ei3 = jnp.concatenate([c0, c1], axis=1)
  src3, dst3 = ei3[0], ei3[1]
  x_pad = jnp.pad(X, ((0, _N_PAD - _N), (0, 0)))

  deg = _make_seg_sum(16, n0, n1)(dst3)

  xwp, dinv = pl.pallas_call(
      _enc1_body,
      out_shape=(
          jax.ShapeDtypeStruct((_N_PAD, _H1), jnp.float32),
          jax.ShapeDtypeStruct((_N_PAD, 16), jnp.float32),
      ),
  )(x_pad, W1, deg)

  s1 = _make_seg_sum(_H1, n0, n1)(src3, dst3, xwp)

  t2 = pl.pallas_call(
      _enc2_body,
      out_shape=jax.ShapeDtypeStruct((_N_PAD, _ENC), jnp.float32),
  )(s1, xwp, dinv, b1.reshape(1, _H1), W2)

  s2 = _make_seg_sum(_ENC, n0, n1)(src3, dst3, t2)

  z, zt = pl.pallas_call(
      _zfin_body,
      out_shape=(
          jax.ShapeDtypeStruct((_N, _ENC), jnp.float32),
          jax.ShapeDtypeStruct((_ENC, _N), jnp.float32),
      ),
  )(s2, t2, dinv, b2.reshape(1, _ENC))

  y = pl.pallas_call(
      _dec_body,
      grid=(_N // _BM,),
      in_specs=[
          pl.BlockSpec((_BM, _ENC), lambda i: (i, 0)),
          pl.BlockSpec((_ENC, _N), lambda i: (0, 0)),
      ],
      out_specs=pl.BlockSpec((_BM, _N), lambda i: (i, 0)),
      out_shape=jax.ShapeDtypeStruct((_N, _N), jnp.float32),
  )(z, zt)
  return y


# speedup vs baseline: 1.0708x; 1.0184x over previous
import functools

import jax
import jax.numpy as jnp
from jax import lax
from jax.experimental import pallas as pl
from jax.experimental.pallas import tpu as pltpu
from jax.experimental.pallas import tpu_sc as plsc

_N = 10000
_D_IN = 128
_H1 = 64
_ENC = 32

_NC = 2
_NS = 16
_NW = _NC * _NS
_K = 128
_N_PAD = 10112
_RPT = _N_PAD // _NS

def _fill(ref, n_rows, n_vec, value):
  def body(i, _):
    for k in range(n_vec):
      ref[i, pl.ds(k * 16, 16)] = jnp.full((16,), value, jnp.float32)
    return 0
  lax.fori_loop(0, n_rows, body, 0)


def _make_seg_sum(depth, n0, n1):
  gather = depth > 16

  def body(*refs):
    if gather:
      (src3, dst3, table, out, idx_s, idx_d, rows_a, rows_b, zrow, acc,
       sem_a, sem_b) = refs
    else:
      dst3, out, idx_d, rows_a, zrow, acc, sem_a = refs
    c = lax.axis_index("c")
    s = lax.axis_index("s")
    wid = c * _NS + s
    n_vec = depth // 16

    _fill(zrow, _RPT, n_vec, 0.0)
    pltpu.sync_copy(zrow, acc.at[pl.ds(s * _RPT, _RPT)])
    if gather:
      pltpu.sync_copy(src3.at[wid], idx_s)
    else:
      _fill(rows_a, _K, n_vec, 1.0)
    pltpu.sync_copy(dst3.at[wid], idx_d)
    plsc.subcore_barrier()

    if gather:
      def run_pipeline(n_chunks):
        n_half = n_chunks // 2
        pltpu.async_copy(table.at[idx_s.at[0]], rows_a, sem_a)

        def edge_body(jj, _):
          j0 = 2 * jj
          pltpu.make_async_copy(table.at[idx_s.at[0]], rows_a, sem_a).wait()
          pltpu.async_copy(table.at[idx_s.at[j0 + 1]], rows_b, sem_b)
          pltpu.sync_copy(rows_a, acc.at[idx_d.at[j0]], add=True)
          pltpu.make_async_copy(table.at[idx_s.at[0]], rows_b, sem_b).wait()
          @pl.when(jj < n_half - 1)
          def _():
            pltpu.async_copy(table.at[idx_s.at[j0 + 2]], rows_a, sem_a)
          pltpu.sync_copy(rows_b, acc.at[idx_d.at[j0 + 1]], add=True)
          return 0
        lax.fori_loop(0, n_half, edge_body, 0)

      if n0 == n1:
        run_pipeline(n0)
      else:
        pl.when(c == 0)(lambda: run_pipeline(n0))
        pl.when(c != 0)(lambda: run_pipeline(n1))
    else:
      def run_scatter(n_chunks):
        def edge_body(j, _):
          pltpu.sync_copy(rows_a, acc.at[idx_d.at[j]], add=True)
          return 0
        lax.fori_loop(0, n_chunks, edge_body, 0)

      if n0 == n1:
        run_scatter(n0)
      else:
        pl.when(c == 0)(lambda: run_scatter(n0))
        pl.when(c != 0)(lambda: run_scatter(n1))

    plsc.subcore_barrier()
    pltpu.sync_copy(acc.at[pl.ds(s * _RPT, _RPT)],
                    out.at[c, pl.ds(s * _RPT, _RPT)])

  n_max = max(n0, n1)
  scratch = []
  if gather:
    scratch.append(pltpu.VMEM((n_max, _K), jnp.int32))
  scratch += [
      pltpu.VMEM((n_max, _K), jnp.int32),
      pltpu.VMEM((_K, depth), jnp.float32),
  ]
  if gather:
    scratch.append(pltpu.VMEM((_K, depth), jnp.float32))
  scratch += [
      pltpu.VMEM((_RPT, depth), jnp.float32),
      pltpu.VMEM_SHARED((_N_PAD, depth), jnp.float32),
      pltpu.SemaphoreType.DMA,
  ]
  if gather:
    scratch.append(pltpu.SemaphoreType.DMA)
  mesh = plsc.VectorSubcoreMesh(
      core_axis_name="c", subcore_axis_name="s", num_cores=_NC,
      num_subcores=_NS)
  return pl.kernel(
      body,
      out_type=jax.ShapeDtypeStruct((_NC, _N_PAD, depth), jnp.float32),
      mesh=mesh,
      scratch_types=scratch,
      compiler_params=pltpu.CompilerParams(use_tc_tiling_on_sc=False),
  )


def _enc1_body(x_ref, w_ref, d_ref, xw_ref, dinv_ref):
  d = d_ref[...]
  deg = d[0, :, 0:1] + d[1, :, 0:1] + 1.0
  dinv = 1.0 / jnp.sqrt(deg)
  xw = jnp.dot(x_ref[...], w_ref[...], preferred_element_type=jnp.float32)
  xw_ref[...] = xw * dinv
  dinv_ref[...] = jnp.broadcast_to(dinv, (_N_PAD, 16))


def _enc2_body(s1_ref, xwp_ref, dinv_ref, b1_ref, w2_ref, t2_ref):
  dv = dinv_ref[:, 0:1]
  s1 = s1_ref[...]
  h = dv * (s1[0] + s1[1] + xwp_ref[...]) + b1_ref[...]
  t2_ref[...] = dv * jnp.dot(h, w2_ref[...], preferred_element_type=jnp.float32)


_BM = 512


def _zfin_body(s2_ref, t2_ref, dinv_ref, b2_ref, z_ref, zt_ref):
  dv = dinv_ref[:, 0:1]
  s2 = s2_ref[...]
  z = dv * (s2[0] + s2[1] + t2_ref[...]) + b2_ref[...]
  z = z[:_N]
  z_ref[...] = z
  zt_ref[...] = z.T


def _dec_body(zb_ref, zt_ref, y_ref):
  acc = jnp.dot(zb_ref[...], zt_ref[...], preferred_element_type=jnp.float32)
  y_ref[...] = 1.0 / (1.0 + jnp.exp(-acc))


_C0_SHARE = 0.75


def kernel(X, edge_index, W1, b1, W2, b2):
  E = edge_index.shape[1]
  unit = _NW * _K * 2
  e_pad = unit * ((E + unit - 1) // unit)
  n_tot = e_pad // (_NS * _K)
  n0 = 2 * int(round(_C0_SHARE * n_tot / 2))
  n0 = min(max(n0, 2), n_tot - 2)
  n1 = n_tot - n0

  ei_pad = jnp.pad(edge_index, ((0, 0), (0, e_pad - E)), constant_values=_N)
  chunks = ei_pad.reshape(2, _NS * n_tot, _K)
  c0 = chunks[:, : _NS * n0].reshape(2, _NS, n0, _K)
  c1 = chunks[:, _NS * n0 :].reshape(2, _NS, n1, _K)
  c1 = jnp.pad(c1, ((0, 0), (0, 0), (0, n0 - n1), (0, 0)),
               constant_values=_N)
  ei3 = jnp.concatenate([c0, c1], axis=1)
  src3, dst3 = ei3[0], ei3[1]
  x_pad = jnp.pad(X, ((0, _N_PAD - _N), (0, 0)))

  deg = _make_seg_sum(16, n0, n1)(dst3)

  xwp, dinv = pl.pallas_call(
      _enc1_body,
      out_shape=(
          jax.ShapeDtypeStruct((_N_PAD, _H1), jnp.float32),
          jax.ShapeDtypeStruct((_N_PAD, 16), jnp.float32),
      ),
  )(x_pad, W1, deg)

  s1 = _make_seg_sum(_H1, n0, n1)(src3, dst3, xwp)

  t2 = pl.pallas_call(
      _enc2_body,
      out_shape=jax.ShapeDtypeStruct((_N_PAD, _ENC), jnp.float32),
  )(s1, xwp, dinv, b1.reshape(1, _H1), W2)

  s2 = _make_seg_sum(_ENC, n0, n1)(src3, dst3, t2)

  z, zt = pl.pallas_call(
      _zfin_body,
      out_shape=(
          jax.ShapeDtypeStruct((_N, _ENC), jnp.float32),
          jax.ShapeDtypeStruct((_ENC, _N), jnp.float32),
      ),
  )(s2, t2, dinv, b2.reshape(1, _ENC))

  grid_m = (_N + _BM - 1) // _BM
  y = pl.pallas_call(
      _dec_body,
      grid=(grid_m,),
      in_specs=[
          pl.BlockSpec((_BM, _ENC), lambda i: (i, 0)),
          pl.BlockSpec((_ENC, _N), lambda i: (0, 0)),
      ],
      out_specs=pl.BlockSpec((_BM, _N), lambda i: (i, 0)),
      out_shape=jax.ShapeDtypeStruct((_N, _N), jnp.float32),
  )(z, zt)
  return y
